# baseline (speedup 1.0000x reference)
import functools

import jax
import jax.numpy as jnp
from jax import lax
from jax.experimental import pallas as pl
from jax.experimental.pallas import tpu as pltpu
from jax.experimental.pallas import tpu_sc as plsc

NUM_LAYERS = 12
NUM_TABLES = 3
REPS = NUM_LAYERS // NUM_TABLES


def _sc_lookup(idx, w0, w1, w2):
    (n,) = idx.shape
    _, d = w0.shape

    info = plsc.get_sparse_core_info()
    nc, ns = info.num_cores, info.num_subcores
    nw = nc * ns
    tpw = n // nw
    chunk = 16
    nchunk = tpw // chunk
    nsteps = NUM_TABLES * nchunk

    mesh = plsc.VectorSubcoreMesh(core_axis_name="c", subcore_axis_name="s")

    @functools.partial(
        pl.kernel,
        mesh=mesh,
        out_type=jax.ShapeDtypeStruct((NUM_LAYERS, n, d), jnp.float32),
        scratch_types=[
            pltpu.VMEM((tpw,), jnp.int32),
            pltpu.VMEM((chunk, d), jnp.float32),
            pltpu.VMEM((chunk, d), jnp.float32),
            pltpu.SemaphoreType.DMA,
            pltpu.SemaphoreType.DMA,
            pltpu.SemaphoreType.DMA,
            pltpu.SemaphoreType.DMA,
        ],
    )
    def k(idx_hbm, w0_hbm, w1_hbm, w2_hbm, out_hbm, idx_v, buf0, buf1,
          gsem0, gsem1, ssem0, ssem1):
        wid = lax.axis_index("s") * nc + lax.axis_index("c")
        base = wid * tpw
        pltpu.sync_copy(idx_hbm.at[pl.ds(base, tpw)], idx_v)
        tables = (w0_hbm, w1_hbm, w2_hbm)
        bufs = (buf0, buf1)
        gsems = (gsem0, gsem1)
        ssems = (ssem0, ssem1)

        def gather(i):
            t, g = divmod(i, nchunk)
            s = i % 2
            return pltpu.async_copy(
                tables[t].at[idx_v.at[pl.ds(g * chunk, chunk)]], bufs[s], gsems[s]
            )

        def stores(i):
            t, g = divmod(i, nchunk)
            s = i % 2
            return [
                pltpu.async_copy(
                    bufs[s],
                    out_hbm.at[t + NUM_TABLES * r, pl.ds(base + g * chunk, chunk), :],
                    ssems[s],
                )
                for r in range(REPS)
            ]

        pending_g = gather(0)
        pending_s = [None, None]
        for i in range(nsteps):
            if i + 1 < nsteps:
                if pending_s[(i + 1) % 2] is not None:
                    for c in pending_s[(i + 1) % 2]:
                        c.wait()
                next_g = gather(i + 1)
            pending_g.wait()
            pending_s[i % 2] = stores(i)
            if i + 1 < nsteps:
                pending_g = next_g
        for s in range(2):
            if pending_s[s] is not None:
                for c in pending_s[s]:
                    c.wait()

    return k(idx, w0, w1, w2)


def kernel(input_seq, W0, W1, W2):
    b, s = input_seq.shape
    _, d = W0.shape
    idx = input_seq.reshape(b * s)
    out = _sc_lookup(idx, W0, W1, W2)
    return out.reshape(NUM_LAYERS, b, s, d)


# speedup vs baseline: 1.1016x; 1.1016x over previous
import functools

import jax
import jax.numpy as jnp
from jax import lax
from jax.experimental import pallas as pl
from jax.experimental.pallas import tpu as pltpu
from jax.experimental.pallas import tpu_sc as plsc

NUM_LAYERS = 12
NUM_TABLES = 3
REPS = NUM_LAYERS // NUM_TABLES


def _sc_lookup(idx, w0, w1, w2):
    (n,) = idx.shape
    _, d = w0.shape

    info = plsc.get_sparse_core_info()
    nc, ns = info.num_cores, info.num_subcores
    nw = nc * ns
    tpw = n // nw
    chunk = 64
    nchunk = tpw // chunk
    nsteps = NUM_TABLES * nchunk

    mesh = plsc.VectorSubcoreMesh(core_axis_name="c", subcore_axis_name="s")

    @functools.partial(
        pl.kernel,
        mesh=mesh,
        out_type=jax.ShapeDtypeStruct((NUM_LAYERS, n, d), jnp.float32),
        scratch_types=[
            pltpu.VMEM((tpw,), jnp.int32),
            pltpu.VMEM((chunk, d), jnp.float32),
            pltpu.SemaphoreType.DMA,
            pltpu.SemaphoreType.DMA,
        ],
    )
    def k(idx_hbm, w0_hbm, w1_hbm, w2_hbm, out_hbm, idx_v, buf, gsem, ssem):
        wid = lax.axis_index("s") * nc + lax.axis_index("c")
        base = wid * tpw
        pltpu.sync_copy(idx_hbm.at[pl.ds(base, tpw)], idx_v)
        tables = (w0_hbm, w1_hbm, w2_hbm)

        for i in range(nsteps):
            t, g = divmod(i, nchunk)
            pltpu.async_copy(
                tables[t].at[idx_v.at[pl.ds(g * chunk, chunk)]], buf, gsem
            ).wait()
            pending = [
                pltpu.async_copy(
                    buf,
                    out_hbm.at[t + NUM_TABLES * r, pl.ds(base + g * chunk, chunk), :],
                    ssem,
                )
                for r in range(REPS)
            ]
            for c in pending:
                c.wait()

    return k(idx, w0, w1, w2)


def kernel(input_seq, W0, W1, W2):
    b, s = input_seq.shape
    _, d = W0.shape
    idx = input_seq.reshape(b * s)
    out = _sc_lookup(idx, W0, W1, W2)
    return out.reshape(NUM_LAYERS, b, s, d)


# speedup vs baseline: 1.1037x; 1.0019x over previous
import functools

import jax
import jax.numpy as jnp
from jax import lax
from jax.experimental import pallas as pl
from jax.experimental.pallas import tpu as pltpu
from jax.experimental.pallas import tpu_sc as plsc

NUM_LAYERS = 12
NUM_TABLES = 3
REPS = NUM_LAYERS // NUM_TABLES


def _sc_lookup(idx, w0, w1, w2):
    (n,) = idx.shape
    _, d = w0.shape

    info = plsc.get_sparse_core_info()
    nc, ns = info.num_cores, info.num_subcores
    nw = nc * ns
    tpw = n // nw
    chunk = 96
    splits = ((0, 96), (96, 32))

    mesh = plsc.VectorSubcoreMesh(core_axis_name="c", subcore_axis_name="s")

    @functools.partial(
        pl.kernel,
        mesh=mesh,
        out_type=jax.ShapeDtypeStruct((NUM_LAYERS, n, d), jnp.float32),
        scratch_types=[
            pltpu.VMEM((tpw,), jnp.int32),
            pltpu.VMEM((chunk, d), jnp.float32),
            pltpu.SemaphoreType.DMA,
            pltpu.SemaphoreType.DMA,
        ],
    )
    def k(idx_hbm, w0_hbm, w1_hbm, w2_hbm, out_hbm, idx_v, buf, gsem, ssem):
        wid = lax.axis_index("s") * nc + lax.axis_index("c")
        base = wid * tpw
        pltpu.sync_copy(idx_hbm.at[pl.ds(base, tpw)], idx_v)
        tables = (w0_hbm, w1_hbm, w2_hbm)

        for t in range(NUM_TABLES):
            for off, cs in splits:
                pltpu.async_copy(
                    tables[t].at[idx_v.at[pl.ds(off, cs)]],
                    buf.at[pl.ds(0, cs), :],
                    gsem,
                ).wait()
                pending = [
                    pltpu.async_copy(
                        buf.at[pl.ds(0, cs), :],
                        out_hbm.at[t + NUM_TABLES * r, pl.ds(base + off, cs), :],
                        ssem,
                    )
                    for r in range(REPS)
                ]
                for c in pending:
                    c.wait()

    return k(idx, w0, w1, w2)


def kernel(input_seq, W0, W1, W2):
    b, s = input_seq.shape
    _, d = W0.shape
    idx = input_seq.reshape(b * s)
    out = _sc_lookup(idx, W0, W1, W2)
    return out.reshape(NUM_LAYERS, b, s, d)


# speedup vs baseline: 1.1100x; 1.0058x over previous
import functools

import jax
import jax.numpy as jnp
from jax import lax
from jax.experimental import pallas as pl
from jax.experimental.pallas import tpu as pltpu
from jax.experimental.pallas import tpu_sc as plsc

NUM_LAYERS = 12
NUM_TABLES = 3
REPS = NUM_LAYERS // NUM_TABLES


def _sc_lookup(idx, w0, w1, w2):
    (n,) = idx.shape
    _, d = w0.shape

    info = plsc.get_sparse_core_info()
    nc, ns = info.num_cores, info.num_subcores
    nw = nc * ns
    tpw = n // nw
    chunk = 112
    splits = ((0, 112), (112, 16))

    mesh = plsc.VectorSubcoreMesh(core_axis_name="c", subcore_axis_name="s")

    @functools.partial(
        pl.kernel,
        mesh=mesh,
        out_type=jax.ShapeDtypeStruct((NUM_LAYERS, n, d), jnp.float32),
        scratch_types=[
            pltpu.VMEM((tpw,), jnp.int32),
            pltpu.VMEM((chunk, d), jnp.float32),
            pltpu.SemaphoreType.DMA,
            pltpu.SemaphoreType.DMA,
        ],
    )
    def k(idx_hbm, w0_hbm, w1_hbm, w2_hbm, out_hbm, idx_v, buf, gsem, ssem):
        wid = lax.axis_index("s") * nc + lax.axis_index("c")
        base = wid * tpw
        pltpu.sync_copy(idx_hbm.at[pl.ds(base, tpw)], idx_v)
        tables = (w0_hbm, w1_hbm, w2_hbm)

        for t in range(NUM_TABLES):
            for off, cs in splits:
                pltpu.async_copy(
                    tables[t].at[idx_v.at[pl.ds(off, cs)]],
                    buf.at[pl.ds(0, cs), :],
                    gsem,
                ).wait()
                pending = [
                    pltpu.async_copy(
                        buf.at[pl.ds(0, cs), :],
                        out_hbm.at[t + NUM_TABLES * r, pl.ds(base + off, cs), :],
                        ssem,
                    )
                    for r in range(REPS)
                ]
                for c in pending:
                    c.wait()

    return k(idx, w0, w1, w2)


def kernel(input_seq, W0, W1, W2):
    b, s = input_seq.shape
    _, d = W0.shape
    idx = input_seq.reshape(b * s)
    out = _sc_lookup(idx, W0, W1, W2)
    return out.reshape(NUM_LAYERS, b, s, d)


# speedup vs baseline: 1.1206x; 1.0095x over previous
import functools

import jax
import jax.numpy as jnp
from jax import lax
from jax.experimental import pallas as pl
from jax.experimental.pallas import tpu as pltpu
from jax.experimental.pallas import tpu_sc as plsc

NUM_LAYERS = 12
NUM_TABLES = 3
REPS = NUM_LAYERS // NUM_TABLES


def _sc_lookup(idx, w0, w1, w2):
    (n,) = idx.shape
    _, d = w0.shape

    info = plsc.get_sparse_core_info()
    nc, ns = info.num_cores, info.num_subcores
    nw = nc * ns
    tpw = n // nw
    chunk = 120
    splits = ((0, 120), (120, 8))

    mesh = plsc.VectorSubcoreMesh(core_axis_name="c", subcore_axis_name="s")

    @functools.partial(
        pl.kernel,
        mesh=mesh,
        out_type=jax.ShapeDtypeStruct((NUM_LAYERS, n, d), jnp.float32),
        scratch_types=[
            pltpu.VMEM((tpw,), jnp.int32),
            pltpu.VMEM((chunk, d), jnp.float32),
            pltpu.SemaphoreType.DMA,
            pltpu.SemaphoreType.DMA,
        ],
    )
    def k(idx_hbm, w0_hbm, w1_hbm, w2_hbm, out_hbm, idx_v, buf, gsem, ssem):
        wid = lax.axis_index("s") * nc + lax.axis_index("c")
        base = wid * tpw
        pltpu.sync_copy(idx_hbm.at[pl.ds(base, tpw)], idx_v)
        tables = (w0_hbm, w1_hbm, w2_hbm)

        for t in range(NUM_TABLES):
            for off, cs in splits:
                pltpu.async_copy(
                    tables[t].at[idx_v.at[pl.ds(off, cs)]],
                    buf.at[pl.ds(0, cs), :],
                    gsem,
                ).wait()
                pending = [
                    pltpu.async_copy(
                        buf.at[pl.ds(0, cs), :],
                        out_hbm.at[t + NUM_TABLES * r, pl.ds(base + off, cs), :],
                        ssem,
                    )
                    for r in range(REPS)
                ]
                for c in pending:
                    c.wait()

    return k(idx, w0, w1, w2)


def kernel(input_seq, W0, W1, W2):
    b, s = input_seq.shape
    _, d = W0.shape
    idx = input_seq.reshape(b * s)
    out = _sc_lookup(idx, W0, W1, W2)
    return out.reshape(NUM_LAYERS, b, s, d)
